# baseline (speedup 1.0000x reference)
import jax
import jax.numpy as jnp
from jax import lax
from jax.experimental import pallas as pl
from jax.experimental.pallas import tpu as pltpu
from jax.experimental.pallas import tpu_sc as plsc

_F = 26
_V = 100000
_E = 32
_NC = 2
_NS = 16
_NW = _NC * _NS
_L = 16
_RPW = (_F * _E) // _NW


def _rne_bf16_hi(u):
    bit = lax.shift_right_logical(u, 16) & jnp.int32(1)
    return lax.shift_right_logical(u + jnp.int32(0x7FFF) + bit, 16)


_VA = 49920
_VB = _V - _VA


def _gather_t_body(emb_ref, idx_ref, out_ref, bufa_v, bufb_v, idxrow_v,
                   outw_v, sema, semb):
    batch = idx_ref.shape[1]
    w = lax.axis_index("s") * _NC + lax.axis_index("c")
    start = _RPW * w
    end = start + _RPW
    lanes = lax.iota(jnp.int32, _L)

    def start_a(r):
        return pltpu.async_copy(emb_ref.at[r, pl.ds(0, _VA)], bufa_v, sema)

    def start_b(r):
        return pltpu.async_copy(emb_ref.at[r, pl.ds(_VA, _VB)], bufb_v, semb)

    start_a(start)
    start_b(start)

    def per_row(r, fprev):
        f = r // _E

        @pl.when(f != fprev)
        def _():
            pltpu.sync_copy(idx_ref.at[f], idxrow_v)

        pltpu.make_async_copy(emb_ref.at[r, pl.ds(0, _VA)], bufa_v, sema).wait()

        def g1(i, c):
            pe = _L * 2 * i + 2 * lanes
            ie = plsc.load_gather(idxrow_v, [pe])
            io = plsc.load_gather(idxrow_v, [pe + 1])
            a = plsc.load_gather(bufa_v, [jnp.minimum(ie, _VA - 1)])
            b = plsc.load_gather(bufa_v, [jnp.minimum(io, _VA - 1)])
            ra = _rne_bf16_hi(plsc.bitcast(a, jnp.int32))
            rb = _rne_bf16_hi(plsc.bitcast(b, jnp.int32))
            outw_v[pl.ds(_L * i, _L)] = ra | lax.shift_left(rb, 16)
            return c

        lax.fori_loop(0, batch // (2 * _L), g1, 0)

        @pl.when(r + 1 < end)
        def _():
            start_a(r + 1)

        pltpu.make_async_copy(emb_ref.at[r, pl.ds(_VA, _VB)], bufb_v, semb).wait()

        def g2(i, c):
            pe = _L * 2 * i + 2 * lanes
            ie = plsc.load_gather(idxrow_v, [pe])
            io = plsc.load_gather(idxrow_v, [pe + 1])
            a = plsc.load_gather(bufb_v, [jnp.maximum(ie - _VA, 0)])
            b = plsc.load_gather(bufb_v, [jnp.maximum(io - _VA, 0)])
            ra = _rne_bf16_hi(plsc.bitcast(a, jnp.int32))
            rb = _rne_bf16_hi(plsc.bitcast(b, jnp.int32))
            word = outw_v[pl.ds(_L * i, _L)]
            we = jnp.where(ie >= _VA, ra, word & jnp.int32(0xFFFF))
            wo = jnp.where(io >= _VA, rb, lax.shift_right_logical(word, 16))
            outw_v[pl.ds(_L * i, _L)] = we | lax.shift_left(wo, 16)
            return c

        lax.fori_loop(0, batch // (2 * _L), g2, 0)

        @pl.when(r + 1 < end)
        def _():
            start_b(r + 1)

        pltpu.sync_copy(outw_v, out_ref.at[r])
        return f

    lax.fori_loop(start, end, per_row, jnp.int32(-1))


def _sc_gather_t(emb_t, idx_t):
    batch = idx_t.shape[1]
    kern = pl.kernel(
        _gather_t_body,
        out_type=jax.ShapeDtypeStruct((_F * _E, batch // 2), jnp.int32),
        mesh=plsc.VectorSubcoreMesh(core_axis_name="c", subcore_axis_name="s"),
        scratch_types=[
            pltpu.VMEM((_VA,), jnp.float32),
            pltpu.VMEM((_VB,), jnp.float32),
            pltpu.VMEM((batch,), jnp.int32),
            pltpu.VMEM((batch // 2,), jnp.int32),
            pltpu.SemaphoreType.DMA,
            pltpu.SemaphoreType.DMA,
        ],
        compiler_params=pltpu.CompilerParams(needs_layout_passes=False),
    )
    return kern(emb_t, idx_t)


def _dot0(a, b):
    return lax.dot_general(a, b, (((0,), (0,)), ((), ())),
                           preferred_element_type=jnp.float32)


def _mlp_body(x_ref, xnum_ref, w1a_ref, w1b_ref, b1_ref, w2_ref, b2_ref,
              w3_ref, b3_ref, w4_ref, b4_ref, out_ref):
    x32 = x_ref[...]
    bbw = x32.shape[1]
    xnum = xnum_ref[...]
    outs = []
    for half in (0, 1):
        if half == 0:
            xb = lax.bitcast_convert_type(lax.shift_left(x32, 16), jnp.float32)
            xn = xnum[:, :bbw]
        else:
            xb = lax.bitcast_convert_type(x32 & jnp.int32(-65536), jnp.float32)
            xn = xnum[:, bbw:]
        h = _dot0(w1a_ref[...], xb.astype(jnp.bfloat16))
        h += _dot0(w1b_ref[...], xn)
        h = jnp.maximum(h + b1_ref[...], 0.0)
        h = jnp.maximum(
            _dot0(w2_ref[...], h.astype(jnp.bfloat16)) + b2_ref[...], 0.0)
        h = jnp.maximum(
            _dot0(w3_ref[...], h.astype(jnp.bfloat16)) + b3_ref[...], 0.0)
        outs.append(_dot0(w4_ref[...], h) + b4_ref[...])
    out_ref[...] = jnp.concatenate(outs, axis=1)


def _tc_mlp_t(xw, xnum_t, w1a, w1b, b1, w2, b2, w3, b3, w4, b4):
    nw = xw.shape[1]
    bbw = nw // 2
    bb = 2 * bbw
    full = lambda a: pl.BlockSpec(a.shape, lambda i: (0, 0))
    return pl.pallas_call(
        _mlp_body,
        grid=(nw // bbw,),
        in_specs=[
            pl.BlockSpec((xw.shape[0], bbw), lambda i: (0, i)),
            pl.BlockSpec((xnum_t.shape[0], bb), lambda i: (0, i)),
            full(w1a), full(w1b), full(b1), full(w2), full(b2),
            full(w3), full(b3), full(w4), full(b4),
        ],
        out_specs=pl.BlockSpec((1, bb), lambda i: (0, i)),
        out_shape=jax.ShapeDtypeStruct((1, 2 * nw), jnp.float32),
    )(xw, xnum_t, w1a, w1b, b1, w2, b2, w3, b3, w4, b4)


def kernel(x_cat, x_num, emb, W1, b1, W2, b2, W3, b3, W4, b4):
    batch = x_cat.shape[0]
    emb_t = jnp.transpose(emb, (0, 2, 1)).reshape(_F * _E, _V)
    idx_t = jnp.transpose(x_cat.astype(jnp.int32))
    xw = _sc_gather_t(emb_t, idx_t)

    nb = 2
    jcols = batch // (2 * nb)
    xnum_t = jnp.pad(jnp.transpose(x_num), ((0, 3), (0, 0)))
    xnum_p = (xnum_t.reshape(-1, nb, jcols, 2)
              .transpose(0, 1, 3, 2).reshape(-1, batch))
    w1a = W1[:_F * _E].astype(jnp.bfloat16)
    w1b = jnp.pad(W1[_F * _E:], ((0, 3), (0, 0)))
    out_t = _tc_mlp_t(
        xw, xnum_p, w1a, w1b,
        b1.reshape(-1, 1), W2.astype(jnp.bfloat16), b2.reshape(-1, 1),
        W3.astype(jnp.bfloat16), b3.reshape(-1, 1), W4, b4.reshape(-1, 1))
    return (out_t.reshape(nb, 2, jcols).transpose(0, 2, 1)
            .reshape(batch, 1))
